# Initial kernel scaffold; baseline (speedup 1.0000x reference)
import jax
import jax.numpy as jnp
from jax.experimental import pallas as pl


def kernel(x, epoch):
    raise NotImplementedError("write your pallas kernel here")


# speedup vs baseline: 82.8438x; 82.8438x over previous
import jax
import jax.numpy as jnp
from jax import lax
from jax.experimental import pallas as pl
from jax.experimental.pallas import tpu as pltpu
from jax.experimental.pallas import tpu_sc as plsc

BASE_RATIO = 0.25
FINAL_RATIO = 0.5
EPOCHS_TOTAL = 100

B0, N0, C0, L = 8, 16, 32, 8192
R = B0 * N0 * C0
NLANE = 16
NCH = L // NLANE
NCORES = 2
NSUB = 16
NW = NCORES * NSUB
RPW = R // NW

MASK31 = 0x7FFFFFFF


def _scan256(hist_ref, k, iota16):
    def it(i, carry):
        above, bstar, krem = carry
        i2 = 15 - i
        v = hist_ref[pl.ds(i2 * NLANE, NLANE)]
        suf_incl = lax.rev(plsc.cumsum(lax.rev(v, (0,))), (0,))
        s = above + (suf_incl - v)
        hit = (s < k) & (s + v >= k)
        bstar = jnp.maximum(
            bstar, jnp.max(jnp.where(hit, i2 * NLANE + iota16, -1)))
        krem = jnp.maximum(krem, jnp.max(jnp.where(hit, k - s, -1)))
        return (above + jnp.max(suf_incl), bstar, krem)

    _, bstar, krem = lax.fori_loop(
        0, 16, it, (jnp.int32(0), jnp.int32(-1), jnp.int32(-1)))
    return bstar, krem


def _sc_body(x_hbm, k_hbm, out_hbm,
             xin0, xin1, mout0, mout1, bits, cbuf, kvm, hist_a, hist_b,
             insem0, insem1, outsem0, outsem1):
    wid = lax.axis_index("s") * NCORES + lax.axis_index("c")
    base = wid * RPW
    pltpu.sync_copy(k_hbm, kvm)
    k0 = jnp.max(kvm[...])
    k0 = jnp.minimum(jnp.maximum(k0, jnp.int32(1)), jnp.int32(L))
    iota16 = lax.iota(jnp.int32, NLANE)
    ones16 = jnp.ones((NLANE,), jnp.int32)
    zeros16 = jnp.zeros((NLANE,), jnp.int32)
    insems = (insem0, insem1)
    outsems = (outsem0, outsem1)
    xins = (xin0, xin1)
    mouts = (mout0, mout1)

    pltpu.async_copy(x_hbm.at[base], xin0, insem0)

    def row_pair(r2, carry):
        for s in range(2):
            r = r2 * 2 + s

            @pl.when(r + 1 < RPW)
            def _start_next():
                pltpu.async_copy(
                    x_hbm.at[base + r + 1], xins[1 - s], insems[1 - s])

            pltpu.make_async_copy(
                x_hbm.at[base + r], xins[s], insems[s]).wait()

            @pl.when(r >= 2)
            def _drain_out():
                pltpu.make_async_copy(
                    mouts[s], out_hbm.at[base + r - 2], outsems[s]).wait()

            xrow = xins[s]
            mrow = mouts[s]

            for i in range(16):
                hist_a[pl.ds(i * NLANE, NLANE)] = zeros16
                hist_b[pl.ds(i * NLANE, NLANE)] = zeros16

            def p1(c, u):
                v = xrow[pl.ds(c * NLANE, NLANE)]
                b = plsc.bitcast(v, jnp.int32) & MASK31
                bits[pl.ds(c * NLANE, NLANE)] = b
                plsc.addupdate_scatter(hist_a, [b >> 23], ones16)
                return u

            lax.fori_loop(0, NCH, p1, 0)
            estar, k1 = _scan256(hist_a, k0, iota16)

            def p2(c, u):
                b = bits[pl.ds(c * NLANE, NLANE)]
                m = (b >> 23) == estar
                plsc.addupdate_scatter(
                    hist_b, [(b >> 15) & 255], ones16, mask=m)
                return u

            lax.fori_loop(0, NCH, p2, 0)
            b2star, k2 = _scan256(hist_b, k1, iota16)
            p16 = (estar << 8) | b2star

            def p3(c, off):
                b = bits[pl.ds(c * NLANE, NLANE)]
                m = (b >> 15) == p16
                plsc.store_compressed(cbuf.at[pl.ds(off, NLANE)], b, mask=m)
                return off + jnp.max(plsc.all_reduce_population_count(m))

            mtot = lax.fori_loop(0, NCH, p3, jnp.int32(0))
            cbuf[pl.ds(mtot, NLANE)] = zeros16

            nchc = (mtot + NLANE - 1) >> 4

            def bitit(j, carry2):
                pv, kr = carry2
                jj = 14 - j

                def cit(c, cnt):
                    b = cbuf[pl.ds(c * NLANE, NLANE)]
                    elig = (b >> (jj + 1)) == (pv >> (jj + 1))
                    mm = elig & (((b >> jj) & 1) == 1)
                    return cnt + jnp.max(plsc.all_reduce_population_count(mm))

                cnt = lax.fori_loop(0, nchc, cit, jnp.int32(0))
                take = kr <= cnt
                pv = jnp.where(take, pv | (jnp.int32(1) << jj), pv)
                kr = jnp.where(take, kr, kr - cnt)
                return (pv, kr)

            tbits, _ = lax.fori_loop(0, 15, bitit, (p16 << 15, k2))

            def pm(c, u):
                b = bits[pl.ds(c * NLANE, NLANE)]
                mrow[pl.ds(c * NLANE, NLANE)] = jnp.where(
                    b >= tbits, jnp.float32(1.0), jnp.float32(0.0))
                return u

            lax.fori_loop(0, NCH, pm, 0)
            pltpu.async_copy(mrow, out_hbm.at[base + r], outsems[s])
        return carry

    lax.fori_loop(0, RPW // 2, row_pair, 0)
    pltpu.make_async_copy(
        mout0, out_hbm.at[base + RPW - 2], outsems[0]).wait()
    pltpu.make_async_copy(
        mout1, out_hbm.at[base + RPW - 1], outsems[1]).wait()


_sc_kernel = pl.kernel(
    _sc_body,
    out_type=jax.ShapeDtypeStruct((R, L), jnp.float32),
    mesh=plsc.VectorSubcoreMesh(core_axis_name="c", subcore_axis_name="s"),
    scratch_types=[
        pltpu.VMEM((L,), jnp.float32),
        pltpu.VMEM((L,), jnp.float32),
        pltpu.VMEM((L,), jnp.float32),
        pltpu.VMEM((L,), jnp.float32),
        pltpu.VMEM((L,), jnp.int32),
        pltpu.VMEM((L + NLANE,), jnp.int32),
        pltpu.VMEM((NLANE,), jnp.int32),
        pltpu.VMEM((256,), jnp.int32),
        pltpu.VMEM((256,), jnp.int32),
        pltpu.SemaphoreType.DMA,
        pltpu.SemaphoreType.DMA,
        pltpu.SemaphoreType.DMA,
        pltpu.SemaphoreType.DMA,
    ],
    compiler_params=pltpu.CompilerParams(needs_layout_passes=False),
)


def kernel(x, epoch):
    ratio = BASE_RATIO + (FINAL_RATIO - BASE_RATIO) * jnp.minimum(
        1.0, epoch / (EPOCHS_TOTAL * 0.8))
    num_mask = jnp.minimum(jnp.floor(L * ratio).astype(jnp.int32), L)
    karr = jnp.full((NLANE,), 1, jnp.int32) * num_mask
    out = _sc_kernel(x.reshape(R, L), karr)
    return out.reshape(x.shape)
